# baseline (speedup 1.0000x reference)
import jax
import jax.numpy as jnp
from jax import lax
from jax.experimental import pallas as pl
from jax.experimental.pallas import tpu as pltpu
from jax.experimental.pallas import tpu_sc as plsc

C = 6
A = 1024
N = 65536
NS = 16384
NT = 32
RPT = NS // NT
RB = 1024
NSB = NS // RB
NTCB = (N - NS) // RB
G = 8
LW = RPT + 32
NBUF = 6
NIMG = 4
HW = 4096
W = 512
NCHUNK = HW // W
GRID = NIMG * NCHUNK


def _sc_body(src_hbm, lab_hbm, sum_out, cnt_out, labs_v, lists_v, acc_v,
             buf_v, cnt_v, sem0, sem1, sem2, sem3, sem4, sem5):
  cid = lax.axis_index("c")
  sid = lax.axis_index("s")
  tid = cid * 16 + sid
  row0 = tid * RPT

  def zrow(r, c):
    def zcol(j, c2):
      acc_v[r, pl.ds(j * 16, 16)] = jnp.zeros((16,), jnp.float32)
      return c2
    return lax.fori_loop(0, A // 16, zcol, c)
  lax.fori_loop(0, C, zrow, 0)

  pltpu.sync_copy(lab_hbm.at[pl.ds(row0, RPT)], labs_v)

  iota = lax.iota(jnp.int32, 16)

  def build(i, offs):
    lv = labs_v[pl.ds(i * 16, 16)]
    rowvec = iota + (row0 + i * 16)
    dest = jnp.zeros((16,), jnp.int32)
    new = []
    for c in range(C):
      mc = lv == c
      mci = jnp.where(mc, jnp.int32(1), jnp.int32(0))
      pos = plsc.cumsum(mci) - mci + offs[c]
      dest = jnp.where(mc, c * LW + pos, dest)
      new.append(offs[c] + jnp.sum(mci))
    plsc.store_scatter(lists_v, [dest], rowvec)
    return tuple(new)

  offs = lax.fori_loop(0, RPT // 16, build,
                       tuple(jnp.int32(0) for _ in range(C)))

  for c in range(C):
    lists_v[pl.ds(c * LW + offs[c], 16)] = jnp.zeros((16,), jnp.int32)

  B = 16
  sems = (sem0, sem1, sem2, sem3, sem4, sem5)

  def fire(c, t, b):
    pltpu.async_copy(src_hbm.at[lists_v.at[pl.ds(c * LW + t * B, B)]],
                     buf_v.at[b], sems[b])

  def wait(c, t, b):
    pltpu.make_async_copy(src_hbm.at[lists_v.at[pl.ds(c * LW + t * B, B)]],
                          buf_v.at[b], sems[b]).wait()

  def accumulate(c, b, nrows):
    def cb(q, c2):
      base = q * 32
      accs = [acc_v[c, pl.ds(base + j * 16, 16)] for j in range(2)]
      for r in range(nrows):
        for j in range(2):
          accs[j] = accs[j] + buf_v[b, r, pl.ds(base + j * 16, 16)]
      for j in range(2):
        acc_v[c, pl.ds(base + j * 16, 16)] = accs[j]
      return c2
    lax.fori_loop(0, A // 32, cb, 0)

  for c in range(C):
    n_c = offs[c]
    nb = n_c // B

    for b0 in range(NBUF):
      @pl.when(nb > b0)
      def _(c=c, b0=b0):
        fire(c, b0, b0)

    def bring(u, carry, c=c, nb=nb):
      for b in range(NBUF):
        t = u * NBUF + b

        @pl.when(t < nb)
        def _(c=c, t=t, b=b, nb=nb):
          wait(c, t, b)
          accumulate(c, b, B)

          @pl.when(t + NBUF < nb)
          def _(c=c, t=t, b=b):
            fire(c, t + NBUF, b)
      return carry

    lax.fori_loop(0, (nb + NBUF - 1) // NBUF, bring, 0)

    rem = n_c - nb * B

    @pl.when(rem > 0)
    def _(c=c, nb=nb, rem=rem):
      pltpu.sync_copy(src_hbm.at[lists_v.at[pl.ds(c * LW + nb * B, B)]],
                      buf_v.at[0])

      def addrow(k, cr, c=c):
        def addcol(j, cr2):
          acc_v[c, pl.ds(j * 16, 16)] += buf_v[0, k, pl.ds(j * 16, 16)]
          return cr2
        return lax.fori_loop(0, A // 16, addcol, cr)
      lax.fori_loop(0, rem, addrow, 0)

  pltpu.sync_copy(acc_v, sum_out.at[tid])
  for c in range(C):
    cnt_v[c, :] = jnp.broadcast_to(offs[c], (16,)).astype(jnp.float32)
  pltpu.sync_copy(cnt_v, cnt_out.at[tid])


def _sc_segment_sums(srcfeat, labels):
  mesh = plsc.VectorSubcoreMesh(core_axis_name="c", subcore_axis_name="s")
  fn = pl.kernel(
      _sc_body,
      out_type=[
          jax.ShapeDtypeStruct((NT, C, A), jnp.float32),
          jax.ShapeDtypeStruct((NT, C, 16), jnp.float32),
      ],
      mesh=mesh,
      scratch_types=[
          pltpu.VMEM((RPT,), jnp.int32),
          pltpu.VMEM((C * LW,), jnp.int32),
          pltpu.VMEM((C, A), jnp.float32),
          pltpu.VMEM((NBUF, 16, A), jnp.float32),
          pltpu.VMEM((C, 16), jnp.float32),
          pltpu.SemaphoreType.DMA,
          pltpu.SemaphoreType.DMA,
          pltpu.SemaphoreType.DMA,
          pltpu.SemaphoreType.DMA,
          pltpu.SemaphoreType.DMA,
          pltpu.SemaphoreType.DMA,
      ],
      compiler_params=pltpu.CompilerParams(needs_layout_passes=False),
  )
  return fn(srcfeat, labels)


def _tc_body(f_ref, o_ref, ofull_ref, proto_ref,
             loss_ref, yn_scr, m_scr, z_scr, w_scr, s1_smem):
  pid = pl.program_id(0)

  @pl.when(pid == 0)
  def _():
    p = proto_ref[...]
    nrm = jnp.sqrt(jnp.sum(p * p, axis=1, keepdims=True))
    yn_scr[...] = p / jnp.maximum(nrm, 1e-12)
    oo = ofull_ref[...]
    m6 = jnp.max(jnp.max(oo, axis=2), axis=0)
    m_scr[...] = jnp.broadcast_to(m6[:, None], (C, 128))
    e = jnp.exp(oo - m6[None, :, None])
    z6 = jnp.sum(jnp.sum(e, axis=2), axis=0)
    z_scr[...] = jnp.broadcast_to(z6[:, None], (C, 128))
    w_scr[...] = jnp.zeros((C, 128), jnp.float32)
    s1_smem[0] = 0.0

  f = f_ref[0]
  ssq = jnp.sum(f * f, axis=0, keepdims=True)
  inv = 1.0 / jnp.maximum(jnp.sqrt(ssq), 1e-12)
  ct = lax.dot_general(
      yn_scr[...], f, (((1,), (0,)), ((), ())),
      preferred_element_type=jnp.float32,
      precision=lax.Precision.HIGHEST)
  ctn = ct * inv

  o = o_ref[0]
  mx = jnp.max(o, axis=0, keepdims=True)
  e1 = jnp.exp(o - mx)
  s1 = e1 / jnp.sum(e1, axis=0, keepdims=True)
  s1_smem[0] += jnp.sum(s1 * ctn)

  e2 = jnp.exp(o - m_scr[:, 0:1])
  w_scr[:, 0:1] += jnp.sum(e2 * ctn, axis=1, keepdims=True)

  @pl.when(pid == GRID - 1)
  def _():
    r = w_scr[:, 0:1] / z_scr[:, 0:1]
    loss = 2.0 - s1_smem[0] / float(NIMG * HW) - jnp.sum(r) / float(C)
    loss_ref[...] = jnp.reshape(loss, (1, 1))


def _tcseg_body(f_ref, lab_ref, sum_ref, cnt_ref, acc_scr, cnt_scr):
  pid = pl.program_id(0)

  @pl.when(pid == 0)
  def _():
    acc_scr[...] = jnp.zeros((8, A), jnp.float32)
    cnt_scr[...] = jnp.zeros((8, 128), jnp.float32)

  lab = lab_ref[0]
  cls = jax.lax.broadcasted_iota(jnp.int32, (8, RB), 0)
  onehot = jnp.where(cls == lab, 1.0, 0.0)
  part = lax.dot_general(
      onehot, f_ref[...], (((1,), (0,)), ((), ())),
      preferred_element_type=jnp.float32,
      precision=lax.Precision.HIGHEST)
  acc_scr[...] += part
  cnt_scr[:, 0:1] += jnp.sum(onehot, axis=1, keepdims=True)

  @pl.when(pid == NTCB - 1)
  def _():
    sum_ref[...] = acc_scr[...]
    cnt_ref[...] = cnt_scr[...]


def _tc_segment_sums(srcfeat, labels3):
  fn = pl.pallas_call(
      _tcseg_body,
      grid=(NTCB,),
      in_specs=[
          pl.BlockSpec((RB, A), lambda s: (NSB + s, 0)),
          pl.BlockSpec((1, 1, RB), lambda s: (NSB + s, 0, 0)),
      ],
      out_specs=[
          pl.BlockSpec((8, A), lambda s: (0, 0)),
          pl.BlockSpec((8, 128), lambda s: (0, 0)),
      ],
      out_shape=[
          jax.ShapeDtypeStruct((8, A), jnp.float32),
          jax.ShapeDtypeStruct((8, 128), jnp.float32),
      ],
      scratch_shapes=[
          pltpu.VMEM((8, A), jnp.float32),
          pltpu.VMEM((8, 128), jnp.float32),
      ],
  )
  return fn(srcfeat, labels3)


def _combine_body(psum_ref, pcnt_ref, tsum_ref, tcnt_ref, mean_ref):
  sums = tsum_ref[0:C, :]
  for i in range(NT):
    sums = sums + psum_ref[i]
  cnt = tcnt_ref[0:C, 0:1]
  for i in range(NT):
    cnt = cnt + pcnt_ref[i, :, 0:1]
  amount = jnp.where(cnt == 0.0, 1.0, cnt)
  mean_ref[...] = sums / amount


def _combine(psum, pcnt, tsum, tcnt):
  fn = pl.pallas_call(
      _combine_body,
      out_shape=jax.ShapeDtypeStruct((C, A), jnp.float32),
  )
  return fn(psum, pcnt, tsum, tcnt)


def _tc_contrastive(tarfeat3, tarout3, Proto):
  fn = pl.pallas_call(
      _tc_body,
      grid=(GRID,),
      in_specs=[
          pl.BlockSpec((1, A, W), lambda s: (s // NCHUNK, 0, s % NCHUNK)),
          pl.BlockSpec((1, C, W), lambda s: (s // NCHUNK, 0, s % NCHUNK)),
          pl.BlockSpec((NIMG, C, HW), lambda s: (0, 0, 0)),
          pl.BlockSpec((C, A), lambda s: (0, 0)),
      ],
      out_specs=pl.BlockSpec((1, 1), lambda s: (0, 0)),
      out_shape=jax.ShapeDtypeStruct((1, 1), jnp.float32),
      scratch_shapes=[
          pltpu.VMEM((C, A), jnp.float32),
          pltpu.VMEM((C, 128), jnp.float32),
          pltpu.VMEM((C, 128), jnp.float32),
          pltpu.VMEM((C, 128), jnp.float32),
          pltpu.SMEM((1,), jnp.float32),
      ],
  )
  return fn(tarfeat3, tarout3, tarout3, Proto)


def kernel(Proto, srcfeat, tarfeat, tarout, labels):
  psum, pcnt = _sc_segment_sums(srcfeat, labels)
  tarfeat3 = tarfeat.reshape(NIMG, A, HW)
  tarout3 = tarout.reshape(NIMG, C, HW)
  loss = _tc_contrastive(tarfeat3, tarout3, Proto)
  tsum, tcnt = _tc_segment_sums(srcfeat, labels.reshape(N // RB, 1, RB))
  mean = _combine(psum, pcnt, tsum, tcnt)
  return (loss[0, 0], mean)


# speedup vs baseline: 1.1969x; 1.1969x over previous
import jax
import jax.numpy as jnp
from jax import lax
from jax.experimental import pallas as pl
from jax.experimental.pallas import tpu as pltpu
from jax.experimental.pallas import tpu_sc as plsc

C = 6
A = 1024
N = 65536
NS = 65536
NT = 32
RPT = NS // NT
RB = 1024
NSB = NS // RB
NTCB = (N - NS) // RB
G = 8
LW = RPT + 32
NBUF = 6
NIMG = 4
HW = 4096
W = 512
NCHUNK = HW // W
GRID = NIMG * NCHUNK


def _sc_body(src_hbm, lab_hbm, sum_out, cnt_out, labs_v, lists_v, acc_v,
             buf_v, cnt_v, sem0, sem1, sem2, sem3, sem4, sem5):
  cid = lax.axis_index("c")
  sid = lax.axis_index("s")
  tid = cid * 16 + sid
  row0 = tid * RPT

  def zrow(r, c):
    def zcol(j, c2):
      acc_v[r, pl.ds(j * 16, 16)] = jnp.zeros((16,), jnp.float32)
      return c2
    return lax.fori_loop(0, A // 16, zcol, c)
  lax.fori_loop(0, C, zrow, 0)

  pltpu.sync_copy(lab_hbm.at[pl.ds(row0, RPT)], labs_v)

  iota = lax.iota(jnp.int32, 16)

  def build(i, offs):
    lv = labs_v[pl.ds(i * 16, 16)]
    rowvec = iota + (row0 + i * 16)
    dest = jnp.zeros((16,), jnp.int32)
    new = []
    for c in range(C):
      mc = lv == c
      mci = jnp.where(mc, jnp.int32(1), jnp.int32(0))
      pos = plsc.cumsum(mci) - mci + offs[c]
      dest = jnp.where(mc, c * LW + pos, dest)
      new.append(offs[c] + jnp.sum(mci))
    plsc.store_scatter(lists_v, [dest], rowvec)
    return tuple(new)

  offs = lax.fori_loop(0, RPT // 16, build,
                       tuple(jnp.int32(0) for _ in range(C)))

  for c in range(C):
    lists_v[pl.ds(c * LW + offs[c], 16)] = jnp.zeros((16,), jnp.int32)

  B = 16
  sems = (sem0, sem1, sem2, sem3, sem4, sem5)

  def fire(c, t, b):
    pltpu.async_copy(src_hbm.at[lists_v.at[pl.ds(c * LW + t * B, B)]],
                     buf_v.at[b], sems[b])

  def wait(c, t, b):
    pltpu.make_async_copy(src_hbm.at[lists_v.at[pl.ds(c * LW + t * B, B)]],
                          buf_v.at[b], sems[b]).wait()

  def accumulate(c, b, nrows):
    def cb(q, c2):
      base = q * 32
      accs = [acc_v[c, pl.ds(base + j * 16, 16)] for j in range(2)]
      for r in range(nrows):
        for j in range(2):
          accs[j] = accs[j] + buf_v[b, r, pl.ds(base + j * 16, 16)]
      for j in range(2):
        acc_v[c, pl.ds(base + j * 16, 16)] = accs[j]
      return c2
    lax.fori_loop(0, A // 32, cb, 0)

  for c in range(C):
    n_c = offs[c]
    nb = n_c // B

    for b0 in range(NBUF):
      @pl.when(nb > b0)
      def _(c=c, b0=b0):
        fire(c, b0, b0)

    def bring(u, carry, c=c, nb=nb):
      for b in range(NBUF):
        t = u * NBUF + b

        @pl.when(t < nb)
        def _(c=c, t=t, b=b, nb=nb):
          wait(c, t, b)
          accumulate(c, b, B)

          @pl.when(t + NBUF < nb)
          def _(c=c, t=t, b=b):
            fire(c, t + NBUF, b)
      return carry

    lax.fori_loop(0, (nb + NBUF - 1) // NBUF, bring, 0)

    rem = n_c - nb * B

    @pl.when(rem > 0)
    def _(c=c, nb=nb, rem=rem):
      pltpu.sync_copy(src_hbm.at[lists_v.at[pl.ds(c * LW + nb * B, B)]],
                      buf_v.at[0])

      def addrow(k, cr, c=c):
        def addcol(j, cr2):
          acc_v[c, pl.ds(j * 16, 16)] += buf_v[0, k, pl.ds(j * 16, 16)]
          return cr2
        return lax.fori_loop(0, A // 16, addcol, cr)
      lax.fori_loop(0, rem, addrow, 0)

  pltpu.sync_copy(acc_v, sum_out.at[tid])
  for c in range(C):
    cnt_v[c, :] = jnp.broadcast_to(offs[c], (16,)).astype(jnp.float32)
  pltpu.sync_copy(cnt_v, cnt_out.at[tid])


def _sc_segment_sums(srcfeat, labels):
  mesh = plsc.VectorSubcoreMesh(core_axis_name="c", subcore_axis_name="s")
  fn = pl.kernel(
      _sc_body,
      out_type=[
          jax.ShapeDtypeStruct((NT, C, A), jnp.float32),
          jax.ShapeDtypeStruct((NT, C, 16), jnp.float32),
      ],
      mesh=mesh,
      scratch_types=[
          pltpu.VMEM((RPT,), jnp.int32),
          pltpu.VMEM((C * LW,), jnp.int32),
          pltpu.VMEM((C, A), jnp.float32),
          pltpu.VMEM((NBUF, 16, A), jnp.float32),
          pltpu.VMEM((C, 16), jnp.float32),
          pltpu.SemaphoreType.DMA,
          pltpu.SemaphoreType.DMA,
          pltpu.SemaphoreType.DMA,
          pltpu.SemaphoreType.DMA,
          pltpu.SemaphoreType.DMA,
          pltpu.SemaphoreType.DMA,
      ],
      compiler_params=pltpu.CompilerParams(needs_layout_passes=False),
  )
  return fn(srcfeat, labels)


def _tc_body(f_ref, o_ref, ofull_ref, proto_ref,
             loss_ref, yn_scr, m_scr, z_scr, w_scr, s1_smem):
  pid = pl.program_id(0)

  @pl.when(pid == 0)
  def _():
    p = proto_ref[...]
    nrm = jnp.sqrt(jnp.sum(p * p, axis=1, keepdims=True))
    yn_scr[...] = p / jnp.maximum(nrm, 1e-12)
    oo = ofull_ref[...]
    m6 = jnp.max(jnp.max(oo, axis=2), axis=0)
    m_scr[...] = jnp.broadcast_to(m6[:, None], (C, 128))
    e = jnp.exp(oo - m6[None, :, None])
    z6 = jnp.sum(jnp.sum(e, axis=2), axis=0)
    z_scr[...] = jnp.broadcast_to(z6[:, None], (C, 128))
    w_scr[...] = jnp.zeros((C, 128), jnp.float32)
    s1_smem[0] = 0.0

  f = f_ref[0]
  ssq = jnp.sum(f * f, axis=0, keepdims=True)
  inv = 1.0 / jnp.maximum(jnp.sqrt(ssq), 1e-12)
  ct = lax.dot_general(
      yn_scr[...], f, (((1,), (0,)), ((), ())),
      preferred_element_type=jnp.float32,
      precision=lax.Precision.HIGHEST)
  ctn = ct * inv

  o = o_ref[0]
  mx = jnp.max(o, axis=0, keepdims=True)
  e1 = jnp.exp(o - mx)
  s1 = e1 / jnp.sum(e1, axis=0, keepdims=True)
  s1_smem[0] += jnp.sum(s1 * ctn)

  e2 = jnp.exp(o - m_scr[:, 0:1])
  w_scr[:, 0:1] += jnp.sum(e2 * ctn, axis=1, keepdims=True)

  @pl.when(pid == GRID - 1)
  def _():
    r = w_scr[:, 0:1] / z_scr[:, 0:1]
    loss = 2.0 - s1_smem[0] / float(NIMG * HW) - jnp.sum(r) / float(C)
    loss_ref[...] = jnp.reshape(loss, (1, 1))


def _tcseg_body(f_ref, lab_ref, sum_ref, cnt_ref, acc_scr, cnt_scr):
  pid = pl.program_id(0)

  @pl.when(pid == 0)
  def _():
    acc_scr[...] = jnp.zeros((8, A), jnp.float32)
    cnt_scr[...] = jnp.zeros((8, 128), jnp.float32)

  lab = lab_ref[0]
  cls = jax.lax.broadcasted_iota(jnp.int32, (8, RB), 0)
  onehot = jnp.where(cls == lab, 1.0, 0.0)
  part = lax.dot_general(
      onehot, f_ref[...], (((1,), (0,)), ((), ())),
      preferred_element_type=jnp.float32,
      precision=lax.Precision.HIGHEST)
  acc_scr[...] += part
  cnt_scr[:, 0:1] += jnp.sum(onehot, axis=1, keepdims=True)

  @pl.when(pid == NTCB - 1)
  def _():
    sum_ref[...] = acc_scr[...]
    cnt_ref[...] = cnt_scr[...]


def _tc_segment_sums(srcfeat, labels3):
  fn = pl.pallas_call(
      _tcseg_body,
      grid=(NTCB,),
      in_specs=[
          pl.BlockSpec((RB, A), lambda s: (NSB + s, 0)),
          pl.BlockSpec((1, 1, RB), lambda s: (NSB + s, 0, 0)),
      ],
      out_specs=[
          pl.BlockSpec((8, A), lambda s: (0, 0)),
          pl.BlockSpec((8, 128), lambda s: (0, 0)),
      ],
      out_shape=[
          jax.ShapeDtypeStruct((8, A), jnp.float32),
          jax.ShapeDtypeStruct((8, 128), jnp.float32),
      ],
      scratch_shapes=[
          pltpu.VMEM((8, A), jnp.float32),
          pltpu.VMEM((8, 128), jnp.float32),
      ],
  )
  return fn(srcfeat, labels3)


def _combine_body(psum_ref, pcnt_ref, mean_ref):
  sums = psum_ref[0]
  for i in range(1, NT):
    sums = sums + psum_ref[i]
  cnt = pcnt_ref[0, :, 0:1]
  for i in range(1, NT):
    cnt = cnt + pcnt_ref[i, :, 0:1]
  amount = jnp.where(cnt == 0.0, 1.0, cnt)
  mean_ref[...] = sums / amount


def _combine(psum, pcnt):
  fn = pl.pallas_call(
      _combine_body,
      out_shape=jax.ShapeDtypeStruct((C, A), jnp.float32),
  )
  return fn(psum, pcnt)


def _tc_contrastive(tarfeat3, tarout3, Proto):
  fn = pl.pallas_call(
      _tc_body,
      grid=(GRID,),
      in_specs=[
          pl.BlockSpec((1, A, W), lambda s: (s // NCHUNK, 0, s % NCHUNK)),
          pl.BlockSpec((1, C, W), lambda s: (s // NCHUNK, 0, s % NCHUNK)),
          pl.BlockSpec((NIMG, C, HW), lambda s: (0, 0, 0)),
          pl.BlockSpec((C, A), lambda s: (0, 0)),
      ],
      out_specs=pl.BlockSpec((1, 1), lambda s: (0, 0)),
      out_shape=jax.ShapeDtypeStruct((1, 1), jnp.float32),
      scratch_shapes=[
          pltpu.VMEM((C, A), jnp.float32),
          pltpu.VMEM((C, 128), jnp.float32),
          pltpu.VMEM((C, 128), jnp.float32),
          pltpu.VMEM((C, 128), jnp.float32),
          pltpu.SMEM((1,), jnp.float32),
      ],
  )
  return fn(tarfeat3, tarout3, tarout3, Proto)


def kernel(Proto, srcfeat, tarfeat, tarout, labels):
  psum, pcnt = _sc_segment_sums(srcfeat, labels)
  tarfeat3 = tarfeat.reshape(NIMG, A, HW)
  tarout3 = tarout.reshape(NIMG, C, HW)
  loss = _tc_contrastive(tarfeat3, tarout3, Proto)
  mean = _combine(psum, pcnt)
  return (loss[0, 0], mean)
